# Initial kernel scaffold; baseline (speedup 1.0000x reference)
import functools

import jax
import jax.numpy as jnp
from jax import lax
from jax.experimental import pallas as pl
from jax.experimental.pallas import tpu as pltpu
from jax.experimental.pallas import tpu_sc as plsc

NUM_HEADS = 4

_NC = 2
_NS = 16
_NW = _NC * _NS

_GCH = 400
_SCH = 128


def _gather_rows(table, idx):
    m = idx.shape[0]
    d = table.shape[1]
    per_w = m // _NW
    n_ch = per_w // _GCH
    assert per_w % _GCH == 0 and per_w % 8 == 0

    mesh = plsc.VectorSubcoreMesh(core_axis_name="c", subcore_axis_name="s")

    @functools.partial(
        pl.kernel,
        mesh=mesh,
        out_type=jax.ShapeDtypeStruct((m, d), jnp.float32),
        scratch_types=[
            pltpu.VMEM((_GCH,), jnp.int32),
            pltpu.VMEM((_GCH, d), jnp.float32),
            pltpu.SemaphoreType.DMA,
        ],
    )
    def k(table_hbm, idx_hbm, out_hbm, idx_v, rows_v, sem):
        wid = lax.axis_index("s") * _NC + lax.axis_index("c")
        base = wid * per_w

        def body(c, carry):
            off = base + c * _GCH
            pltpu.sync_copy(idx_hbm.at[pl.ds(off, _GCH)], idx_v)
            pltpu.async_copy(table_hbm.at[idx_v], rows_v, sem).wait()
            pltpu.sync_copy(rows_v, out_hbm.at[pl.ds(off, _GCH)])
            return carry

        lax.fori_loop(0, n_ch, body, 0, unroll=False)

    return k(table, idx)


def _scatter_max(msg, s_eid, s_dst, bounds, n_nodes):
    d = msg.shape[1]
    npw = -(-n_nodes // _NW)
    n_vec = d // 16

    mesh = plsc.VectorSubcoreMesh(core_axis_name="c", subcore_axis_name="s")

    @functools.partial(
        pl.kernel,
        mesh=mesh,
        out_type=jax.ShapeDtypeStruct((npw * _NW, d), jnp.float32),
        scratch_types=[
            pltpu.VMEM((npw, d), jnp.float32),
            pltpu.VMEM((_SCH,), jnp.int32),
            pltpu.VMEM((_SCH,), jnp.int32),
            pltpu.VMEM((_SCH, d), jnp.float32),
            pltpu.VMEM((2 * _NW,), jnp.int32),
            pltpu.SemaphoreType.DMA,
        ],
    )
    def k(msg_hbm, eid_hbm, dst_hbm, bounds_hbm, out_hbm,
          acc_v, eid_v, dst_v, rows_v, bounds_v, sem):
        wid = lax.axis_index("s") * _NC + lax.axis_index("c")
        nlo = wid * npw
        pltpu.sync_copy(bounds_hbm, bounds_v)
        lo = bounds_v[wid]
        n_chunks = bounds_v[_NW + wid]

        zero16 = jnp.zeros((16,), jnp.float32)

        def zbody(r, carry):
            for j in range(n_vec):
                acc_v[r, pl.ds(j * 16, 16)] = zero16
            return carry

        lax.fori_loop(0, npw, zbody, 0, unroll=False)

        def chunk_body(c, carry):
            off = lo + c * _SCH
            pltpu.sync_copy(eid_hbm.at[pl.ds(off, _SCH)], eid_v)
            pltpu.sync_copy(dst_hbm.at[pl.ds(off, _SCH)], dst_v)
            pltpu.async_copy(msg_hbm.at[eid_v], rows_v, sem).wait()

            def edge_body(e, c2):
                r = dst_v[e] - nlo

                @pl.when(jnp.logical_and(r >= 0, r < npw))
                def _upd():
                    for j in range(n_vec):
                        sl = pl.ds(j * 16, 16)
                        acc_v[r, sl] = jnp.maximum(acc_v[r, sl], rows_v[e, sl])

                return c2

            lax.fori_loop(0, _SCH, edge_body, 0, unroll=False)
            return carry

        lax.fori_loop(0, n_chunks, chunk_body, 0, unroll=False)
        pltpu.sync_copy(acc_v, out_hbm.at[pl.ds(nlo, npw)])

    return k(msg, s_eid, s_dst, bounds)


def _roll_tree(x, op, width):
    s = NUM_HEADS
    while s < width:
        x = op(x, pltpu.roll(x, s, 1))
        s *= 2
    return x


def _edge_body(xi_ref, xj_ref, ef_ref,
               w1a, w1b, w1c, b1, w2, b2,
               wq, bq, we, be, wv, bv,
               m1q, m1e, mb1, m2, mb2,
               ne_ref, prob_ref, msg_ref, *, relu_out):
    dot = functools.partial(jnp.dot, preferred_element_type=jnp.float32)
    xi = xi_ref[...]
    xj = xj_ref[...]
    ef = ef_ref[...]

    h = jnp.maximum(dot(xi, w1a[...]) + dot(ef, w1b[...]) + dot(xj, w1c[...])
                    + b1[...], 0.0)
    ne = dot(h, w2[...]) + b2[...]
    ne_ref[...] = jnp.maximum(ne, 0.0) if relu_out else ne

    q = jnp.maximum(dot(xi, wq[...]) + bq[...], 0.0)
    ep = jnp.maximum(dot(ef, we[...]) + be[...], 0.0)
    v = jnp.maximum(dot(xj, wv[...]) + bv[...], 0.0)

    h2 = jnp.maximum(dot(q, m1q[...]) + dot(ep, m1e[...]) + mb1[...], 0.0)
    logits = dot(h2, m2[...]) + mb2[...]

    gmax = _roll_tree(logits, jnp.maximum, logits.shape[1])
    ex = jnp.exp(logits - gmax)
    gsum = _roll_tree(ex, jnp.add, logits.shape[1])
    prob = ex / gsum
    prob_ref[...] = prob
    msg_ref[...] = prob * v


def _edge_compute(x_i, x_j, ef, wp, relu_out):
    e = x_i.shape[0]
    dn = x_i.shape[1]
    de = ef.shape[1]
    be_blk = 1000
    grid = e // be_blk

    def full(a):
        return pl.BlockSpec(a.shape, lambda i: (0,) * a.ndim)

    row = lambda w: pl.BlockSpec((be_blk, w), lambda i: (i, 0))

    weights = (wp["w1a"], wp["w1b"], wp["w1c"], wp["b1"], wp["w2"], wp["b2"],
               wp["wq"], wp["bq"], wp["we"], wp["be"], wp["wv"], wp["bv"],
               wp["m1q"], wp["m1e"], wp["mb1"], wp["m2"], wp["mb2"])

    return pl.pallas_call(
        functools.partial(_edge_body, relu_out=relu_out),
        grid=(grid,),
        in_specs=[row(dn), row(dn), row(de)] + [full(w) for w in weights],
        out_specs=[row(de), row(dn), row(dn)],
        out_shape=[
            jax.ShapeDtypeStruct((e, de), jnp.float32),
            jax.ShapeDtypeStruct((e, dn), jnp.float32),
            jax.ShapeDtypeStruct((e, dn), jnp.float32),
        ],
    )(x_i, x_j, ef, *weights)


def _node_body(x_ref, agg_ref, p1a, p1b, pb1, p2, pb2, out_ref, *, relu_out):
    dot = functools.partial(jnp.dot, preferred_element_type=jnp.float32)
    h = jnp.maximum(dot(x_ref[...], p1a[...]) + dot(agg_ref[...], p1b[...])
                    + pb1[...], 0.0)
    o = dot(h, p2[...]) + pb2[...]
    out_ref[...] = jnp.maximum(o, 0.0) if relu_out else o


def _node_update(x, agg, wp, relu_out):
    n, dn = x.shape
    bn = 1000
    grid = n // bn

    def full(a):
        return pl.BlockSpec(a.shape, lambda i: (0,) * a.ndim)

    row = lambda w: pl.BlockSpec((bn, w), lambda i: (i, 0))
    weights = (wp["p1a"], wp["p1b"], wp["pb1"], wp["p2"], wp["pb2"])
    return pl.pallas_call(
        functools.partial(_node_body, relu_out=relu_out),
        grid=(grid,),
        in_specs=[row(dn), row(dn)] + [full(w) for w in weights],
        out_specs=row(dn),
        out_shape=jax.ShapeDtypeStruct((n, dn), jnp.float32),
    )(x, agg, *weights)


def _prep_params(p, dim_node, dim_edge):
    eye = jnp.eye(NUM_HEADS, dtype=jnp.float32)
    w1t = p["nn_edge_W1"].T
    wp = {
        "w1a": w1t[:dim_node],
        "w1b": w1t[dim_node:dim_node + dim_edge],
        "w1c": w1t[dim_node + dim_edge:],
        "b1": p["nn_edge_b1"][None, :],
        "w2": p["nn_edge_W2"].T,
        "b2": p["nn_edge_b2"][None, :],
        "wq": p["proj_q_W"].T,
        "bq": p["proj_q_b"][None, :],
        "we": p["proj_e_W"].T,
        "be": p["proj_e_b"][None, :],
        "wv": p["proj_v_W"].T,
        "bv": p["proj_v_b"][None, :],
    }
    m1 = jnp.kron(p["att_W1"].T, eye)
    wp["m1q"] = m1[:dim_node]
    wp["m1e"] = m1[dim_node:]
    wp["mb1"] = jnp.repeat(p["att_b1"], NUM_HEADS)[None, :]
    wp["m2"] = jnp.kron(p["att_W2"].T, eye)
    wp["mb2"] = jnp.repeat(p["att_b2"], NUM_HEADS)[None, :]
    p1t = p["prop_W1"].T
    wp["p1a"] = p1t[:dim_node]
    wp["p1b"] = p1t[dim_node:]
    wp["pb1"] = p["prop_b1"][None, :]
    wp["p2"] = p["prop_W2"].T
    wp["pb2"] = p["prop_b2"][None, :]
    return wp


def kernel(node_feature, edge_feature, edges_indices, params):
    n_nodes, dim_node = node_feature.shape
    n_edges, dim_edge = edge_feature.shape
    num_layers = len(params)
    d_n = dim_node // NUM_HEADS

    ei0 = edges_indices[0]
    idx_flat = edges_indices.reshape(-1)
    perm = jnp.argsort(ei0).astype(jnp.int32)
    s_dst = jnp.take(ei0, perm)
    npw = -(-n_nodes // _NW)
    node_lo = jnp.arange(_NW, dtype=jnp.int32) * npw
    node_hi = node_lo + npw
    lo = jnp.searchsorted(s_dst, node_lo).astype(jnp.int32)
    hi = jnp.searchsorted(s_dst, node_hi).astype(jnp.int32)
    lo8 = (lo // 8) * 8
    nch = -(-(hi - lo8) // _SCH)
    bounds = jnp.concatenate([lo8, nch]).astype(jnp.int32)
    pad = 2 * _SCH
    s_eid = jnp.concatenate([perm, jnp.zeros((pad,), jnp.int32)])
    s_dstp = jnp.concatenate([s_dst, jnp.full((pad,), 1 << 20, jnp.int32)])

    wps = [_prep_params(p, dim_node, dim_edge) for p in params]

    x, e = node_feature, edge_feature
    probs = []
    for li in range(num_layers):
        last = li == num_layers - 1
        relu_flag = (not last) or num_layers == 1
        gath = _gather_rows(x, idx_flat)
        x_i = gath[:n_edges]
        x_j = gath[n_edges:]
        ne, prob_flat, msg = _edge_compute(x_i, x_j, e, wps[li], relu_flag)
        agg = _scatter_max(msg, s_eid, s_dstp, bounds, n_nodes)[:n_nodes]
        x = _node_update(x, agg, wps[li], relu_flag)
        e = ne
        probs.append(prob_flat.reshape(n_edges, d_n, NUM_HEADS))
    return (x, e, probs)


# speedup vs baseline: 1.2714x; 1.2714x over previous
import functools

import jax
import jax.numpy as jnp
from jax import lax
from jax.experimental import pallas as pl
from jax.experimental.pallas import tpu as pltpu
from jax.experimental.pallas import tpu_sc as plsc

NUM_HEADS = 4

_NC = 2
_NS = 16
_NW = _NC * _NS

_GCH = 400
_SCH = 128


def _gather_rows(table, idx):
    m = idx.shape[0]
    d = table.shape[1]
    per_w = m // _NW
    n_ch = per_w // _GCH
    assert per_w % _GCH == 0 and per_w % 8 == 0

    mesh = plsc.VectorSubcoreMesh(core_axis_name="c", subcore_axis_name="s")

    @functools.partial(
        pl.kernel,
        mesh=mesh,
        out_type=jax.ShapeDtypeStruct((m, d), jnp.float32),
        scratch_types=[
            pltpu.VMEM((_GCH,), jnp.int32),
            pltpu.VMEM((_GCH, d), jnp.float32),
            pltpu.SemaphoreType.DMA,
        ],
    )
    def k(table_hbm, idx_hbm, out_hbm, idx_v, rows_v, sem):
        wid = lax.axis_index("s") * _NC + lax.axis_index("c")
        base = wid * per_w

        def body(c, carry):
            off = base + c * _GCH
            pltpu.sync_copy(idx_hbm.at[pl.ds(off, _GCH)], idx_v)
            pltpu.async_copy(table_hbm.at[idx_v], rows_v, sem).wait()
            pltpu.sync_copy(rows_v, out_hbm.at[pl.ds(off, _GCH)])
            return carry

        lax.fori_loop(0, n_ch, body, 0, unroll=False)

    return k(table, idx)


def _scatter_max(msg, s_eid, s_dst, bounds, n_nodes):
    d = msg.shape[1]
    npw = ((-(-n_nodes // _NW)) + 7) // 8 * 8
    n_vec = d // 16

    mesh = plsc.VectorSubcoreMesh(core_axis_name="c", subcore_axis_name="s")

    @functools.partial(
        pl.kernel,
        mesh=mesh,
        out_type=jax.ShapeDtypeStruct((npw * _NW, d), jnp.float32),
        scratch_types=[
            pltpu.VMEM((npw, d), jnp.float32),
            pltpu.VMEM((_SCH,), jnp.int32),
            pltpu.VMEM((_SCH,), jnp.int32),
            pltpu.VMEM((_SCH, d), jnp.float32),
            pltpu.VMEM((2 * _NW, 16), jnp.int32),
            pltpu.SemaphoreType.DMA,
        ],
    )
    def k(msg_hbm, eid_hbm, dst_hbm, bounds_hbm, out_hbm,
          acc_v, eid_v, dst_v, rows_v, bounds_v, sem):
        wid = lax.axis_index("s") * _NC + lax.axis_index("c")
        nlo = wid * npw
        pltpu.sync_copy(bounds_hbm, bounds_v)
        lo = pl.multiple_of(bounds_v[wid, :][0], 8)
        n_chunks = bounds_v[_NW + wid, :][0]

        zero16 = jnp.zeros((16,), jnp.float32)

        def zbody(r, carry):
            for j in range(n_vec):
                acc_v[r, pl.ds(j * 16, 16)] = zero16
            return carry

        lax.fori_loop(0, npw, zbody, 0, unroll=False)

        def chunk_body(c, carry):
            off = lo + c * _SCH
            pltpu.sync_copy(eid_hbm.at[pl.ds(off, _SCH)], eid_v)
            pltpu.sync_copy(dst_hbm.at[pl.ds(off, _SCH)], dst_v)
            pltpu.async_copy(msg_hbm.at[eid_v], rows_v, sem).wait()

            def grp_body(g, c2):
                dvec = dst_v[pl.ds(g * 16, 16)] - nlo
                for j in range(16):
                    r = dvec[j]

                    @pl.when(jnp.logical_and(r >= 0, r < npw))
                    def _upd():
                        for v in range(n_vec):
                            sl = pl.ds(v * 16, 16)
                            acc_v[r, sl] = jnp.maximum(acc_v[r, sl],
                                                       rows_v[g * 16 + j, sl])

                return c2

            lax.fori_loop(0, _SCH // 16, grp_body, 0, unroll=False)
            return carry

        lax.fori_loop(0, n_chunks, chunk_body, 0, unroll=False)
        pltpu.sync_copy(acc_v, out_hbm.at[pl.ds(nlo, npw)])

    return k(msg, s_eid, s_dst, bounds)


def _roll_tree(x, op, width):
    s = NUM_HEADS
    while s < width:
        x = op(x, pltpu.roll(x, s, 1))
        s *= 2
    return x


def _edge_body(xi_ref, xj_ref, ef_ref,
               w1a, w1b, w1c, b1, w2, b2,
               wq, bq, we, be, wv, bv,
               m1q, m1e, mb1, m2, mb2,
               ne_ref, prob_ref, msg_ref, *, relu_out):
    dot = functools.partial(jnp.dot, preferred_element_type=jnp.float32)
    xi = xi_ref[...]
    xj = xj_ref[...]
    ef = ef_ref[...]

    h = jnp.maximum(dot(xi, w1a[...]) + dot(ef, w1b[...]) + dot(xj, w1c[...])
                    + b1[...], 0.0)
    ne = dot(h, w2[...]) + b2[...]
    ne_ref[...] = jnp.maximum(ne, 0.0) if relu_out else ne

    q = jnp.maximum(dot(xi, wq[...]) + bq[...], 0.0)
    ep = jnp.maximum(dot(ef, we[...]) + be[...], 0.0)
    v = jnp.maximum(dot(xj, wv[...]) + bv[...], 0.0)

    h2 = jnp.maximum(dot(q, m1q[...]) + dot(ep, m1e[...]) + mb1[...], 0.0)
    logits = dot(h2, m2[...]) + mb2[...]

    gmax = _roll_tree(logits, jnp.maximum, logits.shape[1])
    ex = jnp.exp(logits - gmax)
    gsum = _roll_tree(ex, jnp.add, logits.shape[1])
    prob = ex / gsum
    prob_ref[...] = prob
    msg_ref[...] = prob * v


def _edge_compute(x_i, x_j, ef, wp, relu_out):
    e = x_i.shape[0]
    dn = x_i.shape[1]
    de = ef.shape[1]
    be_blk = 1000
    grid = e // be_blk

    def full(a):
        return pl.BlockSpec(a.shape, lambda i: (0,) * a.ndim)

    row = lambda w: pl.BlockSpec((be_blk, w), lambda i: (i, 0))

    weights = (wp["w1a"], wp["w1b"], wp["w1c"], wp["b1"], wp["w2"], wp["b2"],
               wp["wq"], wp["bq"], wp["we"], wp["be"], wp["wv"], wp["bv"],
               wp["m1q"], wp["m1e"], wp["mb1"], wp["m2"], wp["mb2"])

    return pl.pallas_call(
        functools.partial(_edge_body, relu_out=relu_out),
        grid=(grid,),
        in_specs=[row(dn), row(dn), row(de)] + [full(w) for w in weights],
        out_specs=[row(de), row(dn), row(dn)],
        out_shape=[
            jax.ShapeDtypeStruct((e, de), jnp.float32),
            jax.ShapeDtypeStruct((e, dn), jnp.float32),
            jax.ShapeDtypeStruct((e, dn), jnp.float32),
        ],
    )(x_i, x_j, ef, *weights)


def _node_body(x_ref, agg_ref, p1a, p1b, pb1, p2, pb2, out_ref, *, relu_out):
    dot = functools.partial(jnp.dot, preferred_element_type=jnp.float32)
    h = jnp.maximum(dot(x_ref[...], p1a[...]) + dot(agg_ref[...], p1b[...])
                    + pb1[...], 0.0)
    o = dot(h, p2[...]) + pb2[...]
    out_ref[...] = jnp.maximum(o, 0.0) if relu_out else o


def _node_update(x, agg, wp, relu_out):
    n, dn = x.shape
    bn = 1000
    grid = n // bn

    def full(a):
        return pl.BlockSpec(a.shape, lambda i: (0,) * a.ndim)

    row = lambda w: pl.BlockSpec((bn, w), lambda i: (i, 0))
    weights = (wp["p1a"], wp["p1b"], wp["pb1"], wp["p2"], wp["pb2"])
    return pl.pallas_call(
        functools.partial(_node_body, relu_out=relu_out),
        grid=(grid,),
        in_specs=[row(dn), row(dn)] + [full(w) for w in weights],
        out_specs=row(dn),
        out_shape=jax.ShapeDtypeStruct((n, dn), jnp.float32),
    )(x, agg, *weights)


def _prep_params(p, dim_node, dim_edge):
    eye = jnp.eye(NUM_HEADS, dtype=jnp.float32)
    w1t = p["nn_edge_W1"].T
    wp = {
        "w1a": w1t[:dim_node],
        "w1b": w1t[dim_node:dim_node + dim_edge],
        "w1c": w1t[dim_node + dim_edge:],
        "b1": p["nn_edge_b1"][None, :],
        "w2": p["nn_edge_W2"].T,
        "b2": p["nn_edge_b2"][None, :],
        "wq": p["proj_q_W"].T,
        "bq": p["proj_q_b"][None, :],
        "we": p["proj_e_W"].T,
        "be": p["proj_e_b"][None, :],
        "wv": p["proj_v_W"].T,
        "bv": p["proj_v_b"][None, :],
    }
    m1 = jnp.kron(p["att_W1"].T, eye)
    wp["m1q"] = m1[:dim_node]
    wp["m1e"] = m1[dim_node:]
    wp["mb1"] = jnp.repeat(p["att_b1"], NUM_HEADS)[None, :]
    wp["m2"] = jnp.kron(p["att_W2"].T, eye)
    wp["mb2"] = jnp.repeat(p["att_b2"], NUM_HEADS)[None, :]
    p1t = p["prop_W1"].T
    wp["p1a"] = p1t[:dim_node]
    wp["p1b"] = p1t[dim_node:]
    wp["pb1"] = p["prop_b1"][None, :]
    wp["p2"] = p["prop_W2"].T
    wp["pb2"] = p["prop_b2"][None, :]
    return wp


def kernel(node_feature, edge_feature, edges_indices, params):
    n_nodes, dim_node = node_feature.shape
    n_edges, dim_edge = edge_feature.shape
    num_layers = len(params)
    d_n = dim_node // NUM_HEADS

    ei0 = edges_indices[0]
    idx_flat = edges_indices.reshape(-1)
    perm = jnp.argsort(ei0).astype(jnp.int32)
    s_dst = jnp.take(ei0, perm)
    npw = ((-(-n_nodes // _NW)) + 7) // 8 * 8
    node_lo = jnp.arange(_NW, dtype=jnp.int32) * npw
    node_hi = node_lo + npw
    lo = jnp.searchsorted(s_dst, node_lo).astype(jnp.int32)
    hi = jnp.searchsorted(s_dst, node_hi).astype(jnp.int32)
    lo8 = (lo // 8) * 8
    nch = -(-(hi - lo8) // _SCH)
    bounds = jnp.broadcast_to(
        jnp.concatenate([lo8, nch]).astype(jnp.int32)[:, None], (2 * _NW, 16)
    )
    pad = 2 * _SCH
    s_eid = jnp.concatenate([perm, jnp.zeros((pad,), jnp.int32)])
    s_dstp = jnp.concatenate([s_dst, jnp.full((pad,), 1 << 20, jnp.int32)])

    wps = [_prep_params(p, dim_node, dim_edge) for p in params]

    x, e = node_feature, edge_feature
    probs = []
    for li in range(num_layers):
        last = li == num_layers - 1
        relu_flag = (not last) or num_layers == 1
        gath = _gather_rows(x, idx_flat)
        x_i = gath[:n_edges]
        x_j = gath[n_edges:]
        ne, prob_flat, msg = _edge_compute(x_i, x_j, e, wps[li], relu_flag)
        agg = _scatter_max(msg, s_eid, s_dstp, bounds, n_nodes)[:n_nodes]
        x = _node_update(x, agg, wps[li], relu_flag)
        e = ne
        probs.append(prob_flat.reshape(n_edges, d_n, NUM_HEADS))
    return (x, e, probs)


# speedup vs baseline: 1.3023x; 1.0243x over previous
import functools

import jax
import jax.numpy as jnp
from jax import lax
from jax.experimental import pallas as pl
from jax.experimental.pallas import tpu as pltpu
from jax.experimental.pallas import tpu_sc as plsc

NUM_HEADS = 4

_NC = 2
_NS = 16
_NW = _NC * _NS

_GCH = 200
_SBLK = 512
_SCH = 64
_SPC = _SBLK // _SCH


def _gather_rows(table, idx):
    m = idx.shape[0]
    d = table.shape[1]
    per_w = m // _NW
    n_ch = per_w // _GCH
    assert per_w % _GCH == 0 and per_w % 8 == 0 and n_ch % 2 == 0

    mesh = plsc.VectorSubcoreMesh(core_axis_name="c", subcore_axis_name="s")

    @functools.partial(
        pl.kernel,
        mesh=mesh,
        out_type=jax.ShapeDtypeStruct((m, d), jnp.float32),
        scratch_types=[
            pltpu.VMEM((per_w,), jnp.int32),
            pltpu.VMEM((_GCH, d), jnp.float32),
            pltpu.VMEM((_GCH, d), jnp.float32),
            pltpu.SemaphoreType.DMA,
            pltpu.SemaphoreType.DMA,
            pltpu.SemaphoreType.DMA,
            pltpu.SemaphoreType.DMA,
        ],
    )
    def k(table_hbm, idx_hbm, out_hbm, idx_v, rows0, rows1, g0, g1, w0, w1):
        wid = lax.axis_index("s") * _NC + lax.axis_index("c")
        base = wid * per_w
        pltpu.sync_copy(idx_hbm.at[pl.ds(base, per_w)], idx_v)
        bufs = (rows0, rows1)
        gs = (g0, g1)
        ws = (w0, w1)

        def gfire(c, b):
            pltpu.async_copy(
                table_hbm.at[idx_v.at[pl.ds(c * _GCH, _GCH)]], bufs[b], gs[b])

        def gwait(b):
            pltpu.make_async_copy(
                table_hbm.at[idx_v.at[pl.ds(0, _GCH)]], bufs[b], gs[b]).wait()

        def wfire(c, b):
            pltpu.async_copy(
                bufs[b], out_hbm.at[pl.ds(base + c * _GCH, _GCH)], ws[b])

        def wwait(b):
            pltpu.make_async_copy(
                bufs[b], out_hbm.at[pl.ds(base, _GCH)], ws[b]).wait()

        gfire(0, 0)

        def body(c2, carry):
            for b in range(2):
                c = c2 * 2 + b
                gwait(b)
                nb = 1 - b

                @pl.when(c + 1 < n_ch)
                def _fire_next():
                    @pl.when(c >= 1)
                    def _drain():
                        wwait(nb)

                    gfire(c + 1, nb)

                wfire(c, b)
            return carry

        lax.fori_loop(0, n_ch // 2, body, 0, unroll=False)
        wwait(0)
        wwait(1)

    return k(table, idx)


def _scatter_max(msg, s_eid, s_dst, bounds, n_nodes):
    d = msg.shape[1]
    npw = ((-(-n_nodes // _NW)) + 7) // 8 * 8
    n_vec = d // 16

    mesh = plsc.VectorSubcoreMesh(core_axis_name="c", subcore_axis_name="s")

    @functools.partial(
        pl.kernel,
        mesh=mesh,
        out_type=jax.ShapeDtypeStruct((npw * _NW, d), jnp.float32),
        scratch_types=[
            pltpu.VMEM((npw, d), jnp.float32),
            pltpu.VMEM((_SBLK,), jnp.int32),
            pltpu.VMEM((_SBLK,), jnp.int32),
            pltpu.VMEM((_SCH, d), jnp.float32),
            pltpu.VMEM((_SCH, d), jnp.float32),
            pltpu.VMEM((2 * _NW, 16), jnp.int32),
            pltpu.SemaphoreType.DMA,
            pltpu.SemaphoreType.DMA,
        ],
    )
    def k(msg_hbm, eid_hbm, dst_hbm, bounds_hbm, out_hbm,
          acc_v, eid_v, dst_v, rows0, rows1, bounds_v, sem0, sem1):
        wid = lax.axis_index("s") * _NC + lax.axis_index("c")
        nlo = wid * npw
        pltpu.sync_copy(bounds_hbm, bounds_v)
        lo = pl.multiple_of(bounds_v[wid, :][0], 8)
        n_blocks = bounds_v[_NW + wid, :][0]
        bufs = (rows0, rows1)
        sems = (sem0, sem1)

        zero16 = jnp.zeros((16,), jnp.float32)

        def zbody(r, carry):
            for j in range(n_vec):
                acc_v[r, pl.ds(j * 16, 16)] = zero16
            return carry

        lax.fori_loop(0, npw, zbody, 0, unroll=False)

        def gfire(kc, b):
            pltpu.async_copy(
                msg_hbm.at[eid_v.at[pl.ds(kc * _SCH, _SCH)]], bufs[b], sems[b])

        def gwait(b):
            pltpu.make_async_copy(
                msg_hbm.at[eid_v.at[pl.ds(0, _SCH)]], bufs[b], sems[b]).wait()

        def accum(kc, b):
            def grp_body(g, c2):
                dvec = dst_v[pl.ds(kc * _SCH + g * 16, 16)] - nlo
                for j in range(16):
                    r = dvec[j]

                    @pl.when(jnp.logical_and(r >= 0, r < npw))
                    def _upd():
                        for v in range(n_vec):
                            sl = pl.ds(v * 16, 16)
                            acc_v[r, sl] = jnp.maximum(acc_v[r, sl],
                                                       bufs[b][g * 16 + j, sl])

                return c2

            lax.fori_loop(0, _SCH // 16, grp_body, 0, unroll=False)

        def blk_body(c, carry):
            off = lo + c * _SBLK
            pltpu.sync_copy(eid_hbm.at[pl.ds(off, _SBLK)], eid_v)
            pltpu.sync_copy(dst_hbm.at[pl.ds(off, _SBLK)], dst_v)
            gfire(0, 0)
            for kc in range(_SPC):
                b = kc % 2
                gwait(b)
                if kc + 1 < _SPC:
                    gfire(kc + 1, 1 - b)
                accum(kc, b)
            return carry

        lax.fori_loop(0, n_blocks, blk_body, 0, unroll=False)
        pltpu.sync_copy(acc_v, out_hbm.at[pl.ds(nlo, npw)])

    return k(msg, s_eid, s_dst, bounds)


def _roll_tree(x, op, width):
    s = NUM_HEADS
    while s < width:
        x = op(x, pltpu.roll(x, s, 1))
        s *= 2
    return x


def _edge_body(xi_ref, xj_ref, ef_ref,
               w1a, w1b, w1c, b1, w2, b2,
               wq, bq, we, be, wv, bv,
               m1q, m1e, mb1, m2, mb2,
               ne_ref, prob_ref, msg_ref, *, relu_out):
    dot = functools.partial(jnp.dot, preferred_element_type=jnp.float32)
    xi = xi_ref[...]
    xj = xj_ref[...]
    ef = ef_ref[...]

    h = jnp.maximum(dot(xi, w1a[...]) + dot(ef, w1b[...]) + dot(xj, w1c[...])
                    + b1[...], 0.0)
    ne = dot(h, w2[...]) + b2[...]
    ne_ref[...] = jnp.maximum(ne, 0.0) if relu_out else ne

    q = jnp.maximum(dot(xi, wq[...]) + bq[...], 0.0)
    ep = jnp.maximum(dot(ef, we[...]) + be[...], 0.0)
    v = jnp.maximum(dot(xj, wv[...]) + bv[...], 0.0)

    h2 = jnp.maximum(dot(q, m1q[...]) + dot(ep, m1e[...]) + mb1[...], 0.0)
    logits = dot(h2, m2[...]) + mb2[...]

    gmax = _roll_tree(logits, jnp.maximum, logits.shape[1])
    ex = jnp.exp(logits - gmax)
    gsum = _roll_tree(ex, jnp.add, logits.shape[1])
    prob = ex / gsum
    prob_ref[...] = prob
    msg_ref[...] = prob * v


def _edge_compute(x_i, x_j, ef, wp, relu_out):
    e = x_i.shape[0]
    dn = x_i.shape[1]
    de = ef.shape[1]
    be_blk = 1000
    grid = e // be_blk

    def full(a):
        return pl.BlockSpec(a.shape, lambda i: (0,) * a.ndim)

    row = lambda w: pl.BlockSpec((be_blk, w), lambda i: (i, 0))

    weights = (wp["w1a"], wp["w1b"], wp["w1c"], wp["b1"], wp["w2"], wp["b2"],
               wp["wq"], wp["bq"], wp["we"], wp["be"], wp["wv"], wp["bv"],
               wp["m1q"], wp["m1e"], wp["mb1"], wp["m2"], wp["mb2"])

    return pl.pallas_call(
        functools.partial(_edge_body, relu_out=relu_out),
        grid=(grid,),
        in_specs=[row(dn), row(dn), row(de)] + [full(w) for w in weights],
        out_specs=[row(de), row(dn), row(dn)],
        out_shape=[
            jax.ShapeDtypeStruct((e, de), jnp.float32),
            jax.ShapeDtypeStruct((e, dn), jnp.float32),
            jax.ShapeDtypeStruct((e, dn), jnp.float32),
        ],
    )(x_i, x_j, ef, *weights)


def _node_body(x_ref, agg_ref, p1a, p1b, pb1, p2, pb2, out_ref, *, relu_out):
    dot = functools.partial(jnp.dot, preferred_element_type=jnp.float32)
    h = jnp.maximum(dot(x_ref[...], p1a[...]) + dot(agg_ref[...], p1b[...])
                    + pb1[...], 0.0)
    o = dot(h, p2[...]) + pb2[...]
    out_ref[...] = jnp.maximum(o, 0.0) if relu_out else o


def _node_update(x, agg, wp, relu_out):
    n, dn = x.shape
    bn = 1000
    grid = n // bn

    def full(a):
        return pl.BlockSpec(a.shape, lambda i: (0,) * a.ndim)

    row = lambda w: pl.BlockSpec((bn, w), lambda i: (i, 0))
    weights = (wp["p1a"], wp["p1b"], wp["pb1"], wp["p2"], wp["pb2"])
    return pl.pallas_call(
        functools.partial(_node_body, relu_out=relu_out),
        grid=(grid,),
        in_specs=[row(dn), row(dn)] + [full(w) for w in weights],
        out_specs=row(dn),
        out_shape=jax.ShapeDtypeStruct((n, dn), jnp.float32),
    )(x, agg, *weights)


def _prep_params(p, dim_node, dim_edge):
    eye = jnp.eye(NUM_HEADS, dtype=jnp.float32)
    w1t = p["nn_edge_W1"].T
    wp = {
        "w1a": w1t[:dim_node],
        "w1b": w1t[dim_node:dim_node + dim_edge],
        "w1c": w1t[dim_node + dim_edge:],
        "b1": p["nn_edge_b1"][None, :],
        "w2": p["nn_edge_W2"].T,
        "b2": p["nn_edge_b2"][None, :],
        "wq": p["proj_q_W"].T,
        "bq": p["proj_q_b"][None, :],
        "we": p["proj_e_W"].T,
        "be": p["proj_e_b"][None, :],
        "wv": p["proj_v_W"].T,
        "bv": p["proj_v_b"][None, :],
    }
    m1 = jnp.kron(p["att_W1"].T, eye)
    wp["m1q"] = m1[:dim_node]
    wp["m1e"] = m1[dim_node:]
    wp["mb1"] = jnp.repeat(p["att_b1"], NUM_HEADS)[None, :]
    wp["m2"] = jnp.kron(p["att_W2"].T, eye)
    wp["mb2"] = jnp.repeat(p["att_b2"], NUM_HEADS)[None, :]
    p1t = p["prop_W1"].T
    wp["p1a"] = p1t[:dim_node]
    wp["p1b"] = p1t[dim_node:]
    wp["pb1"] = p["prop_b1"][None, :]
    wp["p2"] = p["prop_W2"].T
    wp["pb2"] = p["prop_b2"][None, :]
    return wp


def kernel(node_feature, edge_feature, edges_indices, params):
    n_nodes, dim_node = node_feature.shape
    n_edges, dim_edge = edge_feature.shape
    num_layers = len(params)
    d_n = dim_node // NUM_HEADS

    ei0 = edges_indices[0]
    idx_flat = edges_indices.reshape(-1)
    perm = jnp.argsort(ei0).astype(jnp.int32)
    s_dst = jnp.take(ei0, perm)
    npw = ((-(-n_nodes // _NW)) + 7) // 8 * 8
    node_lo = jnp.arange(_NW, dtype=jnp.int32) * npw
    node_hi = node_lo + npw
    lo = jnp.searchsorted(s_dst, node_lo).astype(jnp.int32)
    hi = jnp.searchsorted(s_dst, node_hi).astype(jnp.int32)
    lo8 = (lo // 8) * 8
    nblk = -(-(hi - lo8) // _SBLK)
    bounds = jnp.broadcast_to(
        jnp.concatenate([lo8, nblk]).astype(jnp.int32)[:, None], (2 * _NW, 16)
    )
    pad = 2 * _SBLK
    s_eid = jnp.concatenate([perm, jnp.zeros((pad,), jnp.int32)])
    s_dstp = jnp.concatenate([s_dst, jnp.full((pad,), 1 << 20, jnp.int32)])

    wps = [_prep_params(p, dim_node, dim_edge) for p in params]

    x, e = node_feature, edge_feature
    probs = []
    for li in range(num_layers):
        last = li == num_layers - 1
        relu_flag = (not last) or num_layers == 1
        gath = _gather_rows(x, idx_flat)
        x_i = gath[:n_edges]
        x_j = gath[n_edges:]
        ne, prob_flat, msg = _edge_compute(x_i, x_j, e, wps[li], relu_flag)
        agg = _scatter_max(msg, s_eid, s_dstp, bounds, n_nodes)[:n_nodes]
        x = _node_update(x, agg, wps[li], relu_flag)
        e = ne
        probs.append(prob_flat.reshape(n_edges, d_n, NUM_HEADS))
    return (x, e, probs)
